# baseline (speedup 1.0000x reference)
import functools

import jax
import jax.numpy as jnp
from jax import lax
from jax.experimental import pallas as pl
from jax.experimental.pallas import tpu as pltpu
from jax.experimental.pallas import tpu_sc as plsc

NUM_SEGMENTS = 16
L = 16


def _phase1(flat, segment_ids, *, num_workers, rows_per_w, block_rows):
    tokens, d = flat.shape
    n_blocks = rows_per_w // block_rows
    n_slices = d // L
    assert n_blocks % 4 == 0
    mesh = plsc.VectorSubcoreMesh(core_axis_name="c", subcore_axis_name="s")

    @functools.partial(
        pl.kernel,
        mesh=mesh,
        out_type=[
            jax.ShapeDtypeStruct((num_workers, NUM_SEGMENTS, d), jnp.float32),
            jax.ShapeDtypeStruct((num_workers, L), jnp.float32),
        ],
        scratch_types=[
            pltpu.VMEM((rows_per_w + L,), jnp.int32),
            pltpu.VMEM((4, block_rows, d), jnp.float32),
            pltpu.VMEM((NUM_SEGMENTS, d), jnp.float32),
            pltpu.VMEM((L,), jnp.float32),
            pltpu.SemaphoreType.DMA,
            pltpu.SemaphoreType.DMA,
            pltpu.SemaphoreType.DMA,
            pltpu.SemaphoreType.DMA,
        ],
    )
    def body(flat_hbm, ids_hbm, psums_hbm, pcnts_hbm,
             ids_v, buf_v, acc_v, cnt_v, sem0, sem1, sem2, sem3):
        cid = lax.axis_index("c")
        scid = lax.axis_index("s")
        wid = scid * 2 + cid
        base = wid * rows_per_w

        pltpu.sync_copy(ids_hbm.at[pl.ds(base, rows_per_w)],
                        ids_v.at[pl.ds(0, rows_per_w)])

        zero = jnp.zeros((L,), jnp.float32)
        iota = lax.iota(jnp.int32, L)

        @pl.loop(0, NUM_SEGMENTS)
        def _(r):
            @pl.loop(0, d, step=L)
            def _(c):
                acc_v[r, pl.ds(c, L)] = zero

        cnt_v[...] = zero

        sems = (sem0, sem1, sem2, sem3)

        def block_copy(b, parity):
            return pltpu.make_async_copy(
                flat_hbm.at[pl.ds(base + b * block_rows, block_rows)],
                buf_v.at[parity],
                sems[parity],
            )

        block_copy(0, 0).start()
        block_copy(1, 1).start()
        block_copy(2, 2).start()
        block_copy(3, 3).start()

        @pl.loop(0, n_blocks, step=4)
        def _(b0):
            for p in range(4):
                b = b0 + p
                block_copy(b, p).wait()
                bbuf = buf_v.at[p]
                first = ids_v[pl.ds(b * block_rows, L)][0]
                last = ids_v[pl.ds(b * block_rows + block_rows - L, L)][L - 1]

                @pl.when(first == last)
                def _(b=b, bbuf=bbuf, first=first):
                    def row(i, accs):
                        return tuple(
                            accs[j] + bbuf[i, pl.ds(j * L, L)]
                            for j in range(n_slices))

                    accs = lax.fori_loop(0, block_rows, row,
                                         (zero,) * n_slices, unroll=1)
                    for j in range(n_slices):
                        plsc.addupdate(acc_v.at[first, pl.ds(j * L, L)],
                                       accs[j])
                    plsc.addupdate(
                        cnt_v.at[pl.ds(0, L)],
                        jnp.where(iota == first, float(block_rows), 0.0))

                @pl.when(first != last)
                def _(b=b, bbuf=bbuf):
                    @pl.loop(0, block_rows)
                    def _(i):
                        seg = ids_v[pl.ds(b * block_rows + i, L)][0]
                        for j in range(n_slices):
                            plsc.addupdate(acc_v.at[seg, pl.ds(j * L, L)],
                                           bbuf[i, pl.ds(j * L, L)])
                        plsc.addupdate(cnt_v.at[pl.ds(0, L)],
                                       jnp.where(iota == seg, 1.0, 0.0))

                @pl.when(b + 4 < n_blocks)
                def _(b=b, p=p):
                    block_copy(b + 4, p).start()

        pltpu.sync_copy(acc_v, psums_hbm.at[wid])
        pltpu.sync_copy(cnt_v, pcnts_hbm.at[wid])

    return body(flat, segment_ids)


def _combine(psums_ref, pcnts_ref, out_ref):
    sums = jnp.sum(psums_ref[...], axis=0)
    cnts = jnp.sum(pcnts_ref[...], axis=0)[:NUM_SEGMENTS]
    out_ref[...] = sums / jnp.maximum(cnts, 1.0)[:, None]


def kernel(flat, segment_ids):
    tokens, d = flat.shape
    num_workers = 32
    rows_per_w = tokens // num_workers
    psums, pcnts = _phase1(flat, segment_ids,
                           num_workers=num_workers,
                           rows_per_w=rows_per_w,
                           block_rows=32)
    out = pl.pallas_call(
        _combine,
        out_shape=jax.ShapeDtypeStruct((NUM_SEGMENTS, d), jnp.float32),
    )(psums, pcnts)
    return out
